# baseline (speedup 1.0000x reference)
import functools

import jax
import jax.numpy as jnp
from jax.experimental import pallas as pl
from jax.experimental.pallas import tpu as pltpu


def _main_body(xk_ref, xo_ref, cat_ref, tab_ref, wk_ref, bk_ref, sel_ref,
               wo_ref, bo_ref, known_ref, obs_ref, *, vocab):
    rblk = xk_ref.shape[0]
    xk = xk_ref[...]
    xo = xo_ref[...]
    iota_v = jax.lax.broadcasted_iota(jnp.int32, (rblk, vocab), 1)
    gs = []
    for j in range(cat_ref.shape[1]):
        idx = cat_ref[:, j:j + 1]
        onehot = (idx == iota_v).astype(jnp.float32)
        gs.append(jnp.dot(onehot, tab_ref[j],
                          preferred_element_type=jnp.float32))
    g = jnp.concatenate(gs, axis=1)
    known = (jnp.dot(xk, wk_ref[...], preferred_element_type=jnp.float32)
             + jnp.dot(g, sel_ref[...], preferred_element_type=jnp.float32)
             + bk_ref[...])
    known_ref[...] = known
    obs_ref[...] = (jnp.dot(xo, wo_ref[...], preferred_element_type=jnp.float32)
                    + bo_ref[...])


def _static_body(idx_ref, tab_ref, out_ref, *, vocab):
    rblk = idx_ref.shape[0]
    iota_v = jax.lax.broadcasted_iota(jnp.int32, (rblk, vocab), 1)
    for i in range(tab_ref.shape[0]):
        onehot = (idx_ref[:, i:i + 1] == iota_v).astype(jnp.float32)
        out_ref[:, i, :] = jnp.dot(onehot, tab_ref[i],
                                   preferred_element_type=jnp.float32)


def kernel(static, known_real, known_categorical, observed, static_tables,
           known_cat_tables, real_W, real_b, obs_W, obs_b):
    B, T, n_real = known_real.shape
    n_obs = observed.shape[-1]
    n_cat = known_categorical.shape[-1]
    n_static = static_tables.shape[0]
    vocab, L = static_tables.shape[1], static_tables.shape[2]
    n_known = n_real + n_cat
    R = B * T

    ck = jnp.arange(L * n_known)
    lk, ik = ck // n_known, ck % n_known
    chan = jnp.arange(n_real)[:, None]
    wk = jnp.where(ik[None, :] == chan, real_W[:, lk], 0.0)
    bk = jnp.where(ik < n_real, real_b[jnp.clip(ik, 0, n_real - 1), lk],
                   0.0)[None, :]
    rsel = jnp.arange(n_cat * L)
    sel = ((ik[None, :] >= n_real)
           & (rsel[:, None] // L == ik[None, :] - n_real)
           & (rsel[:, None] % L == lk[None, :])).astype(jnp.float32)
    co = jnp.arange(L * n_obs)
    lo, io = co // n_obs, co % n_obs
    wo = jnp.where(io[None, :] == chan, obs_W[:, lo], 0.0)
    bo = obs_b[io, lo][None, :]

    xk = known_real.reshape(R, n_real)
    xo = observed.reshape(R, n_obs)
    cat = known_categorical.reshape(R, n_cat).astype(jnp.int32)

    RBLK = 256 if R % 256 == 0 else R
    grid = (R // RBLK,)
    full = lambda shape: pl.BlockSpec(shape, lambda r: (0,) * len(shape))
    known_flat, obs_flat = pl.pallas_call(
        functools.partial(_main_body, vocab=vocab),
        grid=grid,
        in_specs=[
            pl.BlockSpec((RBLK, n_real), lambda r: (r, 0)),
            pl.BlockSpec((RBLK, n_obs), lambda r: (r, 0)),
            pl.BlockSpec((RBLK, n_cat), lambda r: (r, 0)),
            full((n_cat, vocab, L)),
            full((n_real, L * n_known)),
            full((1, L * n_known)),
            full((n_cat * L, L * n_known)),
            full((n_real, L * n_obs)),
            full((1, L * n_obs)),
        ],
        out_specs=[
            pl.BlockSpec((RBLK, L * n_known), lambda r: (r, 0)),
            pl.BlockSpec((RBLK, L * n_obs), lambda r: (r, 0)),
        ],
        out_shape=[
            jax.ShapeDtypeStruct((R, L * n_known), jnp.float32),
            jax.ShapeDtypeStruct((R, L * n_obs), jnp.float32),
        ],
    )(xk, xo, cat, known_cat_tables, wk, bk, sel, wo, bo)

    sidx = static[:, 0, :].astype(jnp.int32)
    SBLK = 256 if B % 256 == 0 else B
    static_embs = pl.pallas_call(
        functools.partial(_static_body, vocab=vocab),
        grid=(B // SBLK,),
        in_specs=[
            pl.BlockSpec((SBLK, n_static), lambda r: (r, 0)),
            full((n_static, vocab, L)),
        ],
        out_specs=pl.BlockSpec((SBLK, n_static, L), lambda r: (r, 0, 0)),
        out_shape=jax.ShapeDtypeStruct((B, n_static, L), jnp.float32),
    )(sidx, static_tables)

    return (static_embs,
            known_flat.reshape(B, T, L, n_known),
            obs_flat.reshape(B, T, L, n_obs))
